# baseline (speedup 1.0000x reference)
import functools

import jax
import jax.numpy as jnp
from jax import lax
from jax.experimental import pallas as pl
from jax.experimental.pallas import tpu as pltpu
from jax.experimental.pallas import tpu_sc as plsc

_BATCH = 16384
_SEQ = 50
_DIM = 64
_WIDE = 2 * _DIM
_ROWS = 1000000
_NUM_IDX = _BATCH * _SEQ
_NC = 2
_NS = 16
_NW = _NC * _NS
_CHUNK = 256
_NBUF = 2
_TIN_B = 8192
_HB = _BATCH // 2


def _transpose_table_tc(tbl_t):
    def body(x_ref, o_ref):
        o_ref[:, :_DIM] = x_ref[...].T

    return pl.pallas_call(
        body,
        grid=(pl.cdiv(_ROWS, _TIN_B),),
        in_specs=[pl.BlockSpec((_DIM, _TIN_B), lambda i: (0, i))],
        out_specs=pl.BlockSpec((_TIN_B, _WIDE), lambda i: (i, 0)),
        out_shape=jax.ShapeDtypeStruct((_ROWS, _WIDE), tbl_t.dtype),
        compiler_params=pltpu.CompilerParams(
            dimension_semantics=("arbitrary",)),
    )(tbl_t)


def _transpose_out_tc(x_wide, s_base, y_prev=None):
    n_seq = x_wide.shape[0] // _HB

    def body(x_ref, *rest):
        o_ref = rest[-1]
        o_ref[0, :, :_HB] = x_ref[:, :_DIM].T
        o_ref[0, :, _HB:] = x_ref[:, _DIM:].T

    operands = (x_wide,) if y_prev is None else (x_wide, y_prev)
    in_specs = [pl.BlockSpec((_HB, _WIDE), lambda s: (s, 0))]
    aliases = {}
    if y_prev is not None:
        in_specs.append(pl.BlockSpec(memory_space=pl.ANY))
        aliases = {1: 0}
    return pl.pallas_call(
        body,
        grid=(n_seq,),
        in_specs=in_specs,
        out_specs=pl.BlockSpec((1, _DIM, _BATCH),
                               lambda s: (s + s_base, 0, 0)),
        out_shape=jax.ShapeDtypeStruct((_SEQ, _DIM, _BATCH), x_wide.dtype),
        input_output_aliases=aliases,
        compiler_params=pltpu.CompilerParams(
            dimension_semantics=("arbitrary",)),
    )(*operands)


def _gather_sc(table, idx_flat):
    n_idx = idx_flat.shape[0]
    per_w = n_idx // _NW
    steps = per_w // _CHUNK
    mesh = plsc.VectorSubcoreMesh(core_axis_name="c", subcore_axis_name="s")

    @functools.partial(
        pl.kernel,
        mesh=mesh,
        out_type=jax.ShapeDtypeStruct((n_idx, _DIM), table.dtype),
        scratch_types=[
            pltpu.VMEM((per_w,), jnp.int32),
            [pltpu.VMEM((_CHUNK, _DIM), table.dtype) for _ in range(_NBUF)],
            [pltpu.SemaphoreType.DMA for _ in range(_NBUF)],
            [pltpu.SemaphoreType.DMA for _ in range(_NBUF)],
        ],
        compiler_params=pltpu.CompilerParams(use_tc_tiling_on_sc=False),
    )
    def gather_kernel(table_hbm, idx_hbm, out_hbm, idx_all, rows, gsem, wsem):
        wid = lax.axis_index("s") * _NC + lax.axis_index("c")
        base = wid * per_w
        pltpu.sync_copy(idx_hbm.at[pl.ds(base, per_w)], idx_all)

        def start_gather(s, b):
            pltpu.async_copy(
                table_hbm.at[idx_all.at[pl.ds(s * _CHUNK, _CHUNK)]],
                rows[b], gsem[b])

        def wait_gather(s, b):
            pltpu.make_async_copy(
                table_hbm.at[idx_all.at[pl.ds(s * _CHUNK, _CHUNK)]],
                rows[b], gsem[b]).wait()

        def start_write(s, b):
            pltpu.async_copy(rows[b], out_hbm.at[pl.ds(base + s * _CHUNK,
                                                       _CHUNK)], wsem[b])

        def wait_write(s, b):
            pltpu.make_async_copy(rows[b],
                                  out_hbm.at[pl.ds(base + s * _CHUNK, _CHUNK)],
                                  wsem[b]).wait()

        for b in range(_NBUF):
            start_gather(b, b)

        @pl.loop(0, steps - _NBUF, step=_NBUF)
        def _(c):
            for b in range(_NBUF):
                s = c + b
                wait_gather(s, b)
                start_write(s, b)
                wait_write(s, b)
                start_gather(s + _NBUF, b)

        for b in range(_NBUF):
            s = steps - _NBUF + b
            wait_gather(s, b)
            start_write(s, b)
            wait_write(s, b)

    return gather_kernel(table, idx_flat)


def kernel(item_indices, item_embeddings):
    tbl_t = jnp.swapaxes(item_embeddings, 0, 1)
    table2 = _transpose_table_tc(tbl_t).reshape(2 * _ROWS, _DIM)
    idx_sm = jnp.swapaxes(item_indices, 0, 1)
    idx_p = jnp.transpose(idx_sm.reshape(_SEQ, 2, _HB), (0, 2, 1))
    idx_p = (idx_p.reshape(_NUM_IDX) * 2).astype(jnp.int32)
    half = _NUM_IDX // 2
    x_a = _gather_sc(table2, idx_p[:half])
    x_b = _gather_sc(table2, idx_p[half:])
    xw_a = x_a.reshape(half // 2, _WIDE)
    xw_b = x_b.reshape(half // 2, _WIDE)
    y0 = _transpose_out_tc(xw_a, 0)
    y = _transpose_out_tc(xw_b, _SEQ // 2, y0)
    return jnp.transpose(y, (2, 0, 1))


# speedup vs baseline: 1.1839x; 1.1839x over previous
import functools

import jax
import jax.numpy as jnp
from jax import lax
from jax.experimental import pallas as pl
from jax.experimental.pallas import tpu as pltpu
from jax.experimental.pallas import tpu_sc as plsc

_BATCH = 16384
_SEQ = 50
_DIM = 64
_WIDE = 2 * _DIM
_ROWS = 1000000
_NUM_IDX = _BATCH * _SEQ
_NC = 2
_NS = 16
_NW = _NC * _NS
_CHUNK = 256
_NBUF = 2
_TIN_B = 8192
_TOUT_B = 8192


def _transpose_table_tc(tbl_t):
    def body(x_ref, o_ref):
        o_ref[:, :_DIM] = x_ref[...].T

    return pl.pallas_call(
        body,
        grid=(pl.cdiv(_ROWS, _TIN_B),),
        in_specs=[pl.BlockSpec((_DIM, _TIN_B), lambda i: (0, i))],
        out_specs=pl.BlockSpec((_TIN_B, _WIDE), lambda i: (i, 0)),
        out_shape=jax.ShapeDtypeStruct((_ROWS, _WIDE), tbl_t.dtype),
        compiler_params=pltpu.CompilerParams(
            dimension_semantics=("arbitrary",)),
    )(tbl_t)


def _transpose_out_tc(x_wide, s_base, y_prev=None):
    n_seq = x_wide.shape[0] // _BATCH

    def body(x_ref, *rest):
        o_ref = rest[-1]
        o_ref[0] = x_ref[:, :_DIM].T

    operands = (x_wide,) if y_prev is None else (x_wide, y_prev)
    in_specs = [pl.BlockSpec((_TOUT_B, _WIDE), lambda s, i: (
        s * (_BATCH // _TOUT_B) + i, 0))]
    aliases = {}
    if y_prev is not None:
        in_specs.append(pl.BlockSpec(memory_space=pl.ANY))
        aliases = {1: 0}
    return pl.pallas_call(
        body,
        grid=(n_seq, _BATCH // _TOUT_B),
        in_specs=in_specs,
        out_specs=pl.BlockSpec((1, _DIM, _TOUT_B),
                               lambda s, i: (s + s_base, 0, i)),
        out_shape=jax.ShapeDtypeStruct((_SEQ, _DIM, _BATCH), x_wide.dtype),
        input_output_aliases=aliases,
        compiler_params=pltpu.CompilerParams(
            dimension_semantics=("arbitrary", "arbitrary")),
    )(*operands)


def _gather_sc(table, idx_flat):
    n_idx = idx_flat.shape[0]
    per_w = n_idx // _NW
    steps = per_w // _CHUNK
    assert steps % _NBUF == 0
    mesh = plsc.VectorSubcoreMesh(core_axis_name="c", subcore_axis_name="s")

    @functools.partial(
        pl.kernel,
        mesh=mesh,
        out_type=jax.ShapeDtypeStruct((n_idx, _WIDE), table.dtype),
        scratch_types=[
            pltpu.VMEM((per_w,), jnp.int32),
            [pltpu.VMEM((_CHUNK, _WIDE), table.dtype) for _ in range(_NBUF)],
            [pltpu.SemaphoreType.DMA for _ in range(_NBUF)],
            [pltpu.SemaphoreType.DMA for _ in range(_NBUF)],
        ],
        compiler_params=pltpu.CompilerParams(use_tc_tiling_on_sc=False),
    )
    def gather_kernel(table_hbm, idx_hbm, out_hbm, idx_all, rows, gsem, wsem):
        wid = lax.axis_index("s") * _NC + lax.axis_index("c")
        base = wid * per_w
        pltpu.sync_copy(idx_hbm.at[pl.ds(base, per_w)], idx_all)

        def start_gather(s, b):
            pltpu.async_copy(
                table_hbm.at[idx_all.at[pl.ds(s * _CHUNK, _CHUNK)]],
                rows[b], gsem[b])

        def wait_gather(s, b):
            pltpu.make_async_copy(
                table_hbm.at[idx_all.at[pl.ds(s * _CHUNK, _CHUNK)]],
                rows[b], gsem[b]).wait()

        def start_write(s, b):
            pltpu.async_copy(rows[b], out_hbm.at[pl.ds(base + s * _CHUNK,
                                                       _CHUNK)], wsem[b])

        def wait_write(s, b):
            pltpu.make_async_copy(rows[b],
                                  out_hbm.at[pl.ds(base + s * _CHUNK, _CHUNK)],
                                  wsem[b]).wait()

        for b in range(_NBUF):
            start_gather(b, b)

        @pl.loop(0, steps - _NBUF, step=_NBUF)
        def _(c):
            for b in range(_NBUF):
                s = c + b
                wait_gather(s, b)
                start_write(s, b)
                wait_write(s, b)
                start_gather(s + _NBUF, b)

        for b in range(_NBUF):
            s = steps - _NBUF + b
            wait_gather(s, b)
            start_write(s, b)
            wait_write(s, b)

    return gather_kernel(table, idx_flat)


def kernel(item_indices, item_embeddings):
    tbl_t = jnp.swapaxes(item_embeddings, 0, 1)
    table_wide = _transpose_table_tc(tbl_t)
    idx_sm = jnp.swapaxes(item_indices, 0, 1).reshape(_NUM_IDX)
    idx_sm = idx_sm.astype(jnp.int32)
    half = _NUM_IDX // 2
    x_a = _gather_sc(table_wide, idx_sm[:half])
    x_b = _gather_sc(table_wide, idx_sm[half:])
    y0 = _transpose_out_tc(x_a, 0)
    y = _transpose_out_tc(x_b, _SEQ // 2, y0)
    return jnp.transpose(y, (2, 0, 1))


# speedup vs baseline: 1.2226x; 1.0327x over previous
import functools

import jax
import jax.numpy as jnp
from jax import lax
from jax.experimental import pallas as pl
from jax.experimental.pallas import tpu as pltpu
from jax.experimental.pallas import tpu_sc as plsc

_BATCH = 16384
_SEQ = 50
_DIM = 64
_WIDE = 2 * _DIM
_ROWS = 1000000
_NUM_IDX = _BATCH * _SEQ
_NC = 2
_NS = 16
_NW = _NC * _NS
_CHUNK = 256
_NBUF = 2
_TIN_B = 16384
_TOUT_B = 16384


def _transpose_table_tc(tbl_t):
    def body(x_ref, o_ref):
        o_ref[:, :_DIM] = x_ref[...].T

    return pl.pallas_call(
        body,
        grid=(pl.cdiv(_ROWS, _TIN_B),),
        in_specs=[pl.BlockSpec((_DIM, _TIN_B), lambda i: (0, i))],
        out_specs=pl.BlockSpec((_TIN_B, _WIDE), lambda i: (i, 0)),
        out_shape=jax.ShapeDtypeStruct((_ROWS, _WIDE), tbl_t.dtype),
        compiler_params=pltpu.CompilerParams(
            dimension_semantics=("arbitrary",)),
    )(tbl_t)


def _transpose_out_tc(x_wide, s_base, y_prev=None):
    n_seq = x_wide.shape[0] // _BATCH

    def body(x_ref, *rest):
        o_ref = rest[-1]
        o_ref[0] = x_ref[:, :_DIM].T

    operands = (x_wide,) if y_prev is None else (x_wide, y_prev)
    in_specs = [pl.BlockSpec((_TOUT_B, _WIDE), lambda s, i: (
        s * (_BATCH // _TOUT_B) + i, 0))]
    aliases = {}
    if y_prev is not None:
        in_specs.append(pl.BlockSpec(memory_space=pl.ANY))
        aliases = {1: 0}
    return pl.pallas_call(
        body,
        grid=(n_seq, _BATCH // _TOUT_B),
        in_specs=in_specs,
        out_specs=pl.BlockSpec((1, _DIM, _TOUT_B),
                               lambda s, i: (s + s_base, 0, i)),
        out_shape=jax.ShapeDtypeStruct((_SEQ, _DIM, _BATCH), x_wide.dtype),
        input_output_aliases=aliases,
        compiler_params=pltpu.CompilerParams(
            dimension_semantics=("arbitrary", "arbitrary")),
    )(*operands)


def _gather_sc(table, idx_flat):
    n_idx = idx_flat.shape[0]
    per_w = n_idx // _NW
    steps = per_w // _CHUNK
    assert steps % _NBUF == 0
    mesh = plsc.VectorSubcoreMesh(core_axis_name="c", subcore_axis_name="s")

    @functools.partial(
        pl.kernel,
        mesh=mesh,
        out_type=jax.ShapeDtypeStruct((n_idx, _WIDE), table.dtype),
        scratch_types=[
            pltpu.VMEM((per_w,), jnp.int32),
            [pltpu.VMEM((_CHUNK, _WIDE), table.dtype) for _ in range(_NBUF)],
            [pltpu.SemaphoreType.DMA for _ in range(_NBUF)],
            [pltpu.SemaphoreType.DMA for _ in range(_NBUF)],
        ],
        compiler_params=pltpu.CompilerParams(use_tc_tiling_on_sc=False),
    )
    def gather_kernel(table_hbm, idx_hbm, out_hbm, idx_all, rows, gsem, wsem):
        wid = lax.axis_index("s") * _NC + lax.axis_index("c")
        base = wid * per_w
        pltpu.sync_copy(idx_hbm.at[pl.ds(base, per_w)], idx_all)

        def start_gather(s, b):
            pltpu.async_copy(
                table_hbm.at[idx_all.at[pl.ds(s * _CHUNK, _CHUNK)]],
                rows[b], gsem[b])

        def wait_gather(s, b):
            pltpu.make_async_copy(
                table_hbm.at[idx_all.at[pl.ds(s * _CHUNK, _CHUNK)]],
                rows[b], gsem[b]).wait()

        def start_write(s, b):
            pltpu.async_copy(rows[b], out_hbm.at[pl.ds(base + s * _CHUNK,
                                                       _CHUNK)], wsem[b])

        def wait_write(s, b):
            pltpu.make_async_copy(rows[b],
                                  out_hbm.at[pl.ds(base + s * _CHUNK, _CHUNK)],
                                  wsem[b]).wait()

        for b in range(_NBUF):
            start_gather(b, b)

        @pl.loop(0, steps - _NBUF, step=_NBUF)
        def _(c):
            for b in range(_NBUF):
                s = c + b
                wait_gather(s, b)
                start_write(s, b)
                wait_write(s, b)
                start_gather(s + _NBUF, b)

        for b in range(_NBUF):
            s = steps - _NBUF + b
            wait_gather(s, b)
            start_write(s, b)
            wait_write(s, b)

    return gather_kernel(table, idx_flat)


def kernel(item_indices, item_embeddings):
    tbl_t = jnp.swapaxes(item_embeddings, 0, 1)
    table_wide = _transpose_table_tc(tbl_t)
    idx_sm = jnp.swapaxes(item_indices, 0, 1).reshape(_NUM_IDX)
    idx_sm = idx_sm.astype(jnp.int32)
    half = _NUM_IDX // 2
    x_a = _gather_sc(table_wide, idx_sm[:half])
    x_b = _gather_sc(table_wide, idx_sm[half:])
    y0 = _transpose_out_tc(x_a, 0)
    y = _transpose_out_tc(x_b, _SEQ // 2, y0)
    return jnp.transpose(y, (2, 0, 1))


# speedup vs baseline: 1.2234x; 1.0007x over previous
import functools

import jax
import jax.numpy as jnp
from jax import lax
from jax.experimental import pallas as pl
from jax.experimental.pallas import tpu as pltpu
from jax.experimental.pallas import tpu_sc as plsc

_BATCH = 16384
_SEQ = 50
_DIM = 64
_WIDE = 2 * _DIM
_ROWS = 1000000
_NUM_IDX = _BATCH * _SEQ
_NC = 2
_NS = 16
_NW = _NC * _NS
_CHUNK = 400
_NBUF = 2
_TIN_B = 16384
_TOUT_B = 16384


def _transpose_table_tc(tbl_t):
    def body(x_ref, o_ref):
        o_ref[:, :_DIM] = x_ref[...].T

    return pl.pallas_call(
        body,
        grid=(pl.cdiv(_ROWS, _TIN_B),),
        in_specs=[pl.BlockSpec((_DIM, _TIN_B), lambda i: (0, i))],
        out_specs=pl.BlockSpec((_TIN_B, _WIDE), lambda i: (i, 0)),
        out_shape=jax.ShapeDtypeStruct((_ROWS, _WIDE), tbl_t.dtype),
        compiler_params=pltpu.CompilerParams(
            dimension_semantics=("arbitrary",)),
    )(tbl_t)


def _transpose_out_tc(x_wide, s_base, y_prev=None):
    n_seq = x_wide.shape[0] // _BATCH

    def body(x_ref, *rest):
        o_ref = rest[-1]
        o_ref[0] = x_ref[:, :_DIM].T

    operands = (x_wide,) if y_prev is None else (x_wide, y_prev)
    in_specs = [pl.BlockSpec((_TOUT_B, _WIDE), lambda s, i: (
        s * (_BATCH // _TOUT_B) + i, 0))]
    aliases = {}
    if y_prev is not None:
        in_specs.append(pl.BlockSpec(memory_space=pl.ANY))
        aliases = {1: 0}
    return pl.pallas_call(
        body,
        grid=(n_seq, _BATCH // _TOUT_B),
        in_specs=in_specs,
        out_specs=pl.BlockSpec((1, _DIM, _TOUT_B),
                               lambda s, i: (s + s_base, 0, i)),
        out_shape=jax.ShapeDtypeStruct((_SEQ, _DIM, _BATCH), x_wide.dtype),
        input_output_aliases=aliases,
        compiler_params=pltpu.CompilerParams(
            dimension_semantics=("arbitrary", "arbitrary")),
    )(*operands)


def _gather_sc(table, idx_flat):
    n_idx = idx_flat.shape[0]
    per_w = n_idx // _NW
    steps = per_w // _CHUNK
    assert steps % _NBUF == 0
    mesh = plsc.VectorSubcoreMesh(core_axis_name="c", subcore_axis_name="s")

    @functools.partial(
        pl.kernel,
        mesh=mesh,
        out_type=jax.ShapeDtypeStruct((n_idx, _WIDE), table.dtype),
        scratch_types=[
            pltpu.VMEM((per_w,), jnp.int32),
            [pltpu.VMEM((_CHUNK, _WIDE), table.dtype) for _ in range(_NBUF)],
            [pltpu.SemaphoreType.DMA for _ in range(_NBUF)],
            [pltpu.SemaphoreType.DMA for _ in range(_NBUF)],
        ],
        compiler_params=pltpu.CompilerParams(use_tc_tiling_on_sc=False),
    )
    def gather_kernel(table_hbm, idx_hbm, out_hbm, idx_all, rows, gsem, wsem):
        wid = lax.axis_index("s") * _NC + lax.axis_index("c")
        base = wid * per_w
        pltpu.sync_copy(idx_hbm.at[pl.ds(base, per_w)], idx_all)

        def start_gather(s, b):
            pltpu.async_copy(
                table_hbm.at[idx_all.at[pl.ds(s * _CHUNK, _CHUNK)]],
                rows[b], gsem[b])

        def wait_gather(s, b):
            pltpu.make_async_copy(
                table_hbm.at[idx_all.at[pl.ds(s * _CHUNK, _CHUNK)]],
                rows[b], gsem[b]).wait()

        def start_write(s, b):
            pltpu.async_copy(rows[b], out_hbm.at[pl.ds(base + s * _CHUNK,
                                                       _CHUNK)], wsem[b])

        def wait_write(s, b):
            pltpu.make_async_copy(rows[b],
                                  out_hbm.at[pl.ds(base + s * _CHUNK, _CHUNK)],
                                  wsem[b]).wait()

        for b in range(_NBUF):
            start_gather(b, b)

        @pl.loop(0, steps - _NBUF, step=_NBUF)
        def _(c):
            for b in range(_NBUF):
                s = c + b
                wait_gather(s, b)
                start_write(s, b)
                wait_write(s, b)
                start_gather(s + _NBUF, b)

        for b in range(_NBUF):
            s = steps - _NBUF + b
            wait_gather(s, b)
            start_write(s, b)
            wait_write(s, b)

    return gather_kernel(table, idx_flat)


def kernel(item_indices, item_embeddings):
    tbl_t = jnp.swapaxes(item_embeddings, 0, 1)
    table_wide = _transpose_table_tc(tbl_t)
    idx_sm = jnp.swapaxes(item_indices, 0, 1).reshape(_NUM_IDX)
    idx_sm = idx_sm.astype(jnp.int32)
    half = _NUM_IDX // 2
    x_a = _gather_sc(table_wide, idx_sm[:half])
    x_b = _gather_sc(table_wide, idx_sm[half:])
    y0 = _transpose_out_tc(x_a, 0)
    y = _transpose_out_tc(x_b, _SEQ // 2, y0)
    return jnp.transpose(y, (2, 0, 1))
